# baseline (speedup 1.0000x reference)
import functools

import jax
import jax.numpy as jnp
from jax import lax
from jax.experimental import pallas as pl
from jax.experimental.pallas import tpu as pltpu
from jax.experimental.pallas import tpu_sc as plsc

_NUM_CORES = 2
_NUM_SUBCORES = 16
_NUM_WORKERS = _NUM_CORES * _NUM_SUBCORES


def kernel(table):
    n, d = table.shape
    rows_per_w = n // _NUM_WORKERS

    mesh = plsc.VectorSubcoreMesh(core_axis_name="c", subcore_axis_name="s")

    @functools.partial(
        pl.kernel,
        mesh=mesh,
        out_type=jax.ShapeDtypeStruct((n, d), table.dtype),
        scratch_types=[pltpu.SemaphoreType.DMA],
    )
    def sc_copy(table_hbm, out_hbm, sem):
        wid = lax.axis_index("s") * _NUM_CORES + lax.axis_index("c")
        base = wid * rows_per_w
        sl = pl.ds(base, rows_per_w)
        pltpu.async_copy(table_hbm.at[sl], out_hbm.at[sl], sem).wait()

    return sc_copy(table)


# speedup vs baseline: 23.8195x; 23.8195x over previous
import functools

import jax
import jax.numpy as jnp
from jax import lax
from jax.experimental import pallas as pl
from jax.experimental.pallas import tpu as pltpu
from jax.experimental.pallas import tpu_sc as plsc

_NUM_CORES = 2
_NUM_SUBCORES = 16
_NUM_WORKERS = _NUM_CORES * _NUM_SUBCORES
_CHUNK_ROWS = 32
_NBUF = 3


def kernel(table):
    n, d = table.shape
    rows_per_w = n // _NUM_WORKERS
    nchunk = rows_per_w // _CHUNK_ROWS

    mesh = plsc.VectorSubcoreMesh(core_axis_name="c", subcore_axis_name="s")

    @functools.partial(
        pl.kernel,
        mesh=mesh,
        out_type=jax.ShapeDtypeStruct((n, d), table.dtype),
        scratch_types=[
            pltpu.VMEM((_NBUF, _CHUNK_ROWS, d), table.dtype),
            pltpu.SemaphoreType.DMA((_NBUF,)),
            pltpu.SemaphoreType.DMA((_NBUF,)),
        ],
    )
    def sc_copy(table_hbm, out_hbm, buf, in_sems, out_sems):
        wid = lax.axis_index("s") * _NUM_CORES + lax.axis_index("c")
        base = wid * rows_per_w

        def in_copy(c, b):
            sl = pl.ds(base + c * _CHUNK_ROWS, _CHUNK_ROWS)
            return pltpu.make_async_copy(table_hbm.at[sl], buf.at[b], in_sems.at[b])

        def out_copy(c, b):
            sl = pl.ds(base + c * _CHUNK_ROWS, _CHUNK_ROWS)
            return pltpu.make_async_copy(buf.at[b], out_hbm.at[sl], out_sems.at[b])

        for c in range(min(_NBUF, nchunk)):
            in_copy(c, c).start()
        for c in range(nchunk):
            b = c % _NBUF
            in_copy(c, b).wait()
            out_copy(c, b).start()
            prev = c - 1
            nxt = prev + _NBUF
            if prev >= 0 and nxt < nchunk:
                out_copy(prev, prev % _NBUF).wait()
                in_copy(nxt, nxt % _NBUF).start()
        for c in range(max(0, nchunk - _NBUF), nchunk):
            out_copy(c, c % _NBUF).wait()

    return sc_copy(table)


# speedup vs baseline: 48.0981x; 2.0193x over previous
import jax
import jax.numpy as jnp
from jax.experimental import pallas as pl
from jax.experimental.pallas import tpu as pltpu

_CHUNK_ROWS = 1024
_NBUF = 4


def _dma_ring(table_hbm, out_hbm, buf, in_sems, out_sems):
    n = table_hbm.shape[0]
    nchunk = n // _CHUNK_ROWS

    def in_copy(c, b):
        sl = pl.ds(c * _CHUNK_ROWS, _CHUNK_ROWS)
        return pltpu.make_async_copy(table_hbm.at[sl], buf.at[b], in_sems.at[b])

    def out_copy(c, b):
        sl = pl.ds(c * _CHUNK_ROWS, _CHUNK_ROWS)
        return pltpu.make_async_copy(buf.at[b], out_hbm.at[sl], out_sems.at[b])

    for c in range(min(_NBUF, nchunk)):
        in_copy(c, c).start()
    for c in range(nchunk):
        b = c % _NBUF
        in_copy(c, b).wait()
        out_copy(c, b).start()
        prev = c - 1
        nxt = prev + _NBUF
        if prev >= 0 and nxt < nchunk:
            out_copy(prev, prev % _NBUF).wait()
            in_copy(nxt, nxt % _NBUF).start()
    for c in range(max(0, nchunk - _NBUF), nchunk):
        out_copy(c, c % _NBUF).wait()


def kernel(table):
    n, d = table.shape
    return pl.pallas_call(
        _dma_ring,
        in_specs=[pl.BlockSpec(memory_space=pl.MemorySpace.ANY)],
        out_specs=pl.BlockSpec(memory_space=pl.MemorySpace.ANY),
        out_shape=jax.ShapeDtypeStruct((n, d), table.dtype),
        scratch_shapes=[
            pltpu.VMEM((_NBUF, _CHUNK_ROWS, d), table.dtype),
            pltpu.SemaphoreType.DMA((_NBUF,)),
            pltpu.SemaphoreType.DMA((_NBUF,)),
        ],
    )(table)


# speedup vs baseline: 48.5008x; 1.0084x over previous
import jax
import jax.numpy as jnp
from jax.experimental import pallas as pl
from jax.experimental.pallas import tpu as pltpu

_CHUNK_ROWS = 2048
_NBUF = 3


def _dma_ring(table_hbm, out_hbm, buf, in_sems, out_sems):
    n = table_hbm.shape[0]
    nchunk = n // _CHUNK_ROWS

    def in_copy(c, b):
        sl = pl.ds(c * _CHUNK_ROWS, _CHUNK_ROWS)
        return pltpu.make_async_copy(table_hbm.at[sl], buf.at[b], in_sems.at[b])

    def out_copy(c, b):
        sl = pl.ds(c * _CHUNK_ROWS, _CHUNK_ROWS)
        return pltpu.make_async_copy(buf.at[b], out_hbm.at[sl], out_sems.at[b])

    for c in range(min(_NBUF, nchunk)):
        in_copy(c, c).start()
    for c in range(nchunk):
        b = c % _NBUF
        in_copy(c, b).wait()
        out_copy(c, b).start()
        prev = c - 1
        nxt = prev + _NBUF
        if prev >= 0 and nxt < nchunk:
            out_copy(prev, prev % _NBUF).wait()
            in_copy(nxt, nxt % _NBUF).start()
    for c in range(max(0, nchunk - _NBUF), nchunk):
        out_copy(c, c % _NBUF).wait()


def kernel(table):
    n, d = table.shape
    return pl.pallas_call(
        _dma_ring,
        in_specs=[pl.BlockSpec(memory_space=pl.MemorySpace.ANY)],
        out_specs=pl.BlockSpec(memory_space=pl.MemorySpace.ANY),
        out_shape=jax.ShapeDtypeStruct((n, d), table.dtype),
        scratch_shapes=[
            pltpu.VMEM((_NBUF, _CHUNK_ROWS, d), table.dtype),
            pltpu.SemaphoreType.DMA((_NBUF,)),
            pltpu.SemaphoreType.DMA((_NBUF,)),
        ],
    )(table)


# speedup vs baseline: 48.7799x; 1.0058x over previous
import jax
import jax.numpy as jnp
from jax.experimental import pallas as pl
from jax.experimental.pallas import tpu as pltpu

_CHUNK_ROWS = 2048
_NBUF = 4


def _dma_ring(table_hbm, out_hbm, buf, in_sems, out_sems):
    n = table_hbm.shape[0]
    nchunk = n // _CHUNK_ROWS

    def in_copy(c, b):
        sl = pl.ds(c * _CHUNK_ROWS, _CHUNK_ROWS)
        return pltpu.make_async_copy(table_hbm.at[sl], buf.at[b], in_sems.at[b])

    def out_copy(c, b):
        sl = pl.ds(c * _CHUNK_ROWS, _CHUNK_ROWS)
        return pltpu.make_async_copy(buf.at[b], out_hbm.at[sl], out_sems.at[b])

    for c in range(min(_NBUF, nchunk)):
        in_copy(c, c).start()
    for c in range(nchunk):
        b = c % _NBUF
        in_copy(c, b).wait()
        out_copy(c, b).start()
        prev = c - 1
        nxt = prev + _NBUF
        if prev >= 0 and nxt < nchunk:
            out_copy(prev, prev % _NBUF).wait()
            in_copy(nxt, nxt % _NBUF).start()
    for c in range(max(0, nchunk - _NBUF), nchunk):
        out_copy(c, c % _NBUF).wait()


def kernel(table):
    n, d = table.shape
    return pl.pallas_call(
        _dma_ring,
        in_specs=[pl.BlockSpec(memory_space=pl.MemorySpace.ANY)],
        out_specs=pl.BlockSpec(memory_space=pl.MemorySpace.ANY),
        out_shape=jax.ShapeDtypeStruct((n, d), table.dtype),
        scratch_shapes=[
            pltpu.VMEM((_NBUF, _CHUNK_ROWS, d), table.dtype),
            pltpu.SemaphoreType.DMA((_NBUF,)),
            pltpu.SemaphoreType.DMA((_NBUF,)),
        ],
    )(table)
